# baseline (speedup 1.0000x reference)
import jax
import jax.numpy as jnp
from jax.experimental import pallas as pl

_N = 16777216
_R, _C = 2048, 8192
_BR = 256


def _copy_patch_kernel(x_ref, o_ref):
    o_ref[...] = x_ref[...]

    @pl.when(pl.program_id(0) == 0)
    def _():
        row = x_ref[0:1, :]
        col = jax.lax.broadcasted_iota(jnp.int32, (1, _C), 1)
        patched = jnp.where(col == 0, 10.0,
                  jnp.where(col == 1, 30.0,
                  jnp.where(col == 2, 20.0,
                  jnp.where(col == 3, 40.0, row))))
        o_ref[0:1, :] = patched


def kernel(data):
    x = data.reshape(_R, _C)
    out = pl.pallas_call(
        _copy_patch_kernel,
        grid=(_R // _BR,),
        in_specs=[pl.BlockSpec((_BR, _C), lambda i: (i, 0))],
        out_specs=pl.BlockSpec((_BR, _C), lambda i: (i, 0)),
        out_shape=jax.ShapeDtypeStruct((_R, _C), jnp.float32),
    )(x)
    return out.reshape(_N)
